# Initial kernel scaffold; baseline (speedup 1.0000x reference)
import jax
import jax.numpy as jnp
from jax.experimental import pallas as pl


def kernel(x, edge_index, edge_weight, hidden_state, W_z, b_z, W_r, b_r, W_h, b_h):
    raise NotImplementedError("write your pallas kernel here")


# speedup vs baseline: 6.7387x; 6.7387x over previous
import functools

import jax
import jax.numpy as jnp
from jax import lax
from jax.experimental import pallas as pl
from jax.experimental.pallas import tpu as pltpu
from jax.experimental.pallas import tpu_sc as plsc

N = 10000
NP = 10240
E = 320000
D = 128
NC = 2
NS = 16
CH = 80
RPT = NP // NS

f32 = jnp.float32
i32 = jnp.int32


def _sc_mesh():
    return plsc.VectorSubcoreMesh(core_axis_name="c", subcore_axis_name="s")


@functools.partial(
    pl.kernel,
    out_type=jax.ShapeDtypeStruct((NC * NS * NP,), f32),
    mesh=_sc_mesh(),
    compiler_params=pltpu.CompilerParams(needs_layout_passes=False),
    scratch_types=[
        pltpu.VMEM((NP,), f32),
        pltpu.VMEM((CH,), i32),
        pltpu.VMEM((CH,), f32),
    ],
)
def _deg_kernel(dst_hbm, ew_hbm, out_hbm, deg_v, dst_v, ew_v):
    c = lax.axis_index("c")
    s = lax.axis_index("s")
    wid = c * NS + s
    zero16 = jnp.zeros((16,), f32)

    def zbody(i, carry):
        deg_v[pl.ds(i * 16, 16)] = zero16
        return carry

    lax.fori_loop(0, NP // 16, zbody, None)

    ept = E // (NC * NS)
    nch = ept // CH
    base_e = wid * ept

    def chunk(ci, carry):
        off = base_e + ci * CH
        pltpu.sync_copy(dst_hbm.at[pl.ds(off, CH)], dst_v)
        pltpu.sync_copy(ew_hbm.at[pl.ds(off, CH)], ew_v)

        def ebody(j, cc):
            idx = dst_v[pl.ds(j * 16, 16)]
            val = ew_v[pl.ds(j * 16, 16)]
            plsc.addupdate_scatter(deg_v, [idx], val)
            return cc

        lax.fori_loop(0, CH // 16, ebody, None)
        return carry

    lax.fori_loop(0, nch, chunk, None)
    pltpu.sync_copy(deg_v, out_hbm.at[pl.ds(wid * NP, NP)])


def _make_agg(base0, count0, base1, count1):
    @functools.partial(
        pl.kernel,
        out_type=(
            jax.ShapeDtypeStruct((NP, D), f32),
            jax.ShapeDtypeStruct((NP, D), f32),
        ),
        mesh=_sc_mesh(),
        compiler_params=pltpu.CompilerParams(needs_layout_passes=False),
        scratch_types=[
            pltpu.VMEM((CH,), i32),
            pltpu.VMEM((CH,), i32),
            pltpu.VMEM((CH,), f32),
            pltpu.VMEM((CH, D), f32),
            pltpu.VMEM_SHARED((NP, D), f32),
        ],
    )
    def agg(t0, t1, src_hbm, dst_hbm, ew_hbm, out0, out1,
            src_v, dst_v, ew_v, rows_v, acc):
        c = lax.axis_index("c")
        s = lax.axis_index("s")
        zero16 = jnp.zeros((16,), f32)

        def zb(i, carry):
            for k in range(D // 16):
                rows_v[i, pl.ds(k * 16, 16)] = zero16
            return carry

        lax.fori_loop(0, CH, zb, None)
        for k in range(RPT // CH):
            pltpu.sync_copy(rows_v, acc.at[pl.ds(s * RPT + k * CH, CH)])
        plsc.subcore_barrier()

        def run(table, out, base, count):
            ept = count // NS
            nch = ept // CH

            def chunk(ci, carry):
                off = base + s * ept + ci * CH
                pltpu.sync_copy(src_hbm.at[pl.ds(off, CH)], src_v)
                pltpu.sync_copy(ew_hbm.at[pl.ds(off, CH)], ew_v)
                pltpu.sync_copy(table.at[src_v], rows_v)

                def ebody(j, cc):
                    w = plsc.load_gather(ew_v, [jnp.zeros((16,), i32) + j])
                    for k in range(D // 16):
                        sl = pl.ds(k * 16, 16)
                        rows_v[j, sl] = rows_v[j, sl] * w
                    return cc

                lax.fori_loop(0, CH, ebody, None)
                pltpu.sync_copy(dst_hbm.at[pl.ds(off, CH)], dst_v)
                pltpu.sync_copy(rows_v, acc.at[dst_v], add=True)
                return carry

            lax.fori_loop(0, nch, chunk, None)
            plsc.subcore_barrier()
            for k in range(RPT // CH):
                r0 = s * RPT + k * CH
                pltpu.sync_copy(acc.at[pl.ds(r0, CH)], rows_v)
                pltpu.sync_copy(rows_v, out.at[pl.ds(r0, CH)])

        pl.when(c == 0)(lambda: run(t0, out0, base0, count0))
        pl.when(c == 1)(lambda: run(t1, out1, base1, count1))

    return agg


_agg_cols = _make_agg(0, E, 0, E)
_agg_edges = _make_agg(0, E // 2, E // 2, E // 2)


def _tc1(degp, x, h):
    def body(degp_ref, x_ref, h_ref, dinv_ref, yx_ref, yh_ref):
        deg = jnp.sum(degp_ref[...], axis=0)[:N] + 1.0
        dinv = lax.rsqrt(deg)[:, None]
        dinv_ref[...] = dinv
        yx_ref[...] = dinv * x_ref[...]
        yh_ref[...] = dinv * h_ref[...]

    return pl.pallas_call(
        body,
        out_shape=(
            jax.ShapeDtypeStruct((N, 1), f32),
            jax.ShapeDtypeStruct((N, D), f32),
            jax.ShapeDtypeStruct((N, D), f32),
        ),
    )(degp, x, h)


_BLK = 1000


def _tc2(aggx, aggh, yx, yh, dinv, h, wz, wr, wh0, bz, br):
    def body(aggx_ref, aggh_ref, yx_ref, yh_ref, dinv_ref, h_ref,
             wz_ref, wr_ref, wh0_ref, bz_ref, br_ref,
             z_ref, yrh_ref, hp_ref):
        dv = dinv_ref[...]
        a_x = dv * (aggx_ref[...] + yx_ref[...])
        a_h = dv * (aggh_ref[...] + yh_ref[...])
        wz_ = wz_ref[...]
        wr_ = wr_ref[...]
        z = jax.nn.sigmoid(
            jnp.dot(a_x, wz_[:D], preferred_element_type=f32)
            + jnp.dot(a_h, wz_[D:], preferred_element_type=f32)
            + bz_ref[...])
        r = jax.nn.sigmoid(
            jnp.dot(a_x, wr_[:D], preferred_element_type=f32)
            + jnp.dot(a_h, wr_[D:], preferred_element_type=f32)
            + br_ref[...])
        z_ref[...] = z
        yrh_ref[...] = dv * (r * h_ref[...])
        hp_ref[...] = jnp.dot(a_x, wh0_ref[...], preferred_element_type=f32)

    grid = (N // _BLK,)
    row_spec = pl.BlockSpec((_BLK, D), lambda i: (i, 0))
    return pl.pallas_call(
        body,
        grid=grid,
        in_specs=[
            row_spec, row_spec, row_spec, row_spec,
            pl.BlockSpec((_BLK, 1), lambda i: (i, 0)),
            row_spec,
            pl.BlockSpec((2 * D, D), lambda i: (0, 0)),
            pl.BlockSpec((2 * D, D), lambda i: (0, 0)),
            pl.BlockSpec((D, D), lambda i: (0, 0)),
            pl.BlockSpec((1, D), lambda i: (0, 0)),
            pl.BlockSpec((1, D), lambda i: (0, 0)),
        ],
        out_specs=(row_spec, row_spec, row_spec),
        out_shape=(
            jax.ShapeDtypeStruct((N, D), f32),
            jax.ShapeDtypeStruct((N, D), f32),
            jax.ShapeDtypeStruct((N, D), f32),
        ),
    )(aggx, aggh, yx, yh, dinv, h, wz, wr, wh0, bz, br)


def _tc3(p0, p1, yrh, dinv, hp, z, h, wh1, bh):
    def body(p0_ref, p1_ref, yrh_ref, dinv_ref, hp_ref, z_ref, h_ref,
             wh1_ref, bh_ref, out_ref):
        dv = dinv_ref[...]
        a_rh = dv * (p0_ref[...] + p1_ref[...] + yrh_ref[...])
        hc = jnp.tanh(
            hp_ref[...]
            + jnp.dot(a_rh, wh1_ref[...], preferred_element_type=f32)
            + bh_ref[...])
        zz = z_ref[...]
        out_ref[...] = zz * h_ref[...] + (1.0 - zz) * hc

    grid = (N // _BLK,)
    row_spec = pl.BlockSpec((_BLK, D), lambda i: (i, 0))
    return pl.pallas_call(
        body,
        grid=grid,
        in_specs=[
            row_spec, row_spec, row_spec,
            pl.BlockSpec((_BLK, 1), lambda i: (i, 0)),
            row_spec, row_spec, row_spec,
            pl.BlockSpec((D, D), lambda i: (0, 0)),
            pl.BlockSpec((1, D), lambda i: (0, 0)),
        ],
        out_specs=row_spec,
        out_shape=jax.ShapeDtypeStruct((N, D), f32),
    )(p0, p1, yrh, dinv, hp, z, h, wh1, bh)


def kernel(x, edge_index, edge_weight, hidden_state, W_z, b_z, W_r, b_r,
           W_h, b_h):
    src = edge_index[0]
    dst = edge_index[1]
    ew = edge_weight

    degp = _deg_kernel(dst, ew).reshape(NC * NS, NP)
    dinv, y_x, y_h = _tc1(degp, x, hidden_state)
    aggx, aggh = _agg_cols(y_x, y_h, src, dst, ew)
    z, y_rh, hpart = _tc2(
        aggx[:N], aggh[:N], y_x, y_h, dinv, hidden_state,
        W_z, W_r, W_h[:D], b_z.reshape(1, D), b_r.reshape(1, D))
    p0, p1 = _agg_edges(y_rh, y_rh, src, dst, ew)
    h_new = _tc3(p0[:N], p1[:N], y_rh, dinv, hpart, z, hidden_state,
                 W_h[D:], b_h.reshape(1, D))
    return h_new


# speedup vs baseline: 13.5903x; 2.0168x over previous
import functools

import jax
import jax.numpy as jnp
from jax import lax
from jax.experimental import pallas as pl
from jax.experimental.pallas import tpu as pltpu
from jax.experimental.pallas import tpu_sc as plsc

N = 10000
NP = 10240
E = 320000
D = 128
NC = 2
NS = 16
CH = 80
RPT = NP // NS

f32 = jnp.float32
i32 = jnp.int32


def _sc_mesh():
    return plsc.VectorSubcoreMesh(core_axis_name="c", subcore_axis_name="s")


_EPT_DEG = E // (NC * NS)


@functools.partial(
    pl.kernel,
    out_type=jax.ShapeDtypeStruct((NC * NS * NP,), f32),
    mesh=_sc_mesh(),
    compiler_params=pltpu.CompilerParams(needs_layout_passes=False),
    scratch_types=[
        pltpu.VMEM((NP,), f32),
        pltpu.VMEM((_EPT_DEG,), i32),
        pltpu.VMEM((_EPT_DEG,), f32),
    ],
)
def _deg_kernel(dst_hbm, ew_hbm, out_hbm, deg_v, dst_v, ew_v):
    c = lax.axis_index("c")
    s = lax.axis_index("s")
    wid = c * NS + s
    zero16 = jnp.zeros((16,), f32)

    def zbody(i, carry):
        deg_v[pl.ds(i * 16, 16)] = zero16
        return carry

    lax.fori_loop(0, NP // 16, zbody, None)

    base_e = wid * _EPT_DEG
    pltpu.sync_copy(dst_hbm.at[pl.ds(base_e, _EPT_DEG)], dst_v)
    pltpu.sync_copy(ew_hbm.at[pl.ds(base_e, _EPT_DEG)], ew_v)

    @plsc.parallel_loop(0, _EPT_DEG // 16, unroll=4)
    def ebody(j):
        idx = dst_v[pl.ds(j * 16, 16)]
        val = ew_v[pl.ds(j * 16, 16)]
        plsc.addupdate_scatter(deg_v, [idx], val)

    pltpu.sync_copy(deg_v, out_hbm.at[pl.ds(wid * NP, NP)])


def _make_agg(ch, edge_split):
    ngroups = NC * NS if edge_split else NS
    ept = E // ngroups
    nch = ept // ch
    nquads = nch // 4
    ntail = nch % 4

    @functools.partial(
        pl.kernel,
        out_type=(
            jax.ShapeDtypeStruct((NP, D), f32),
            jax.ShapeDtypeStruct((NP, D), f32),
        ),
        mesh=_sc_mesh(),
        compiler_params=pltpu.CompilerParams(needs_layout_passes=False),
        scratch_types=[
            pltpu.VMEM((4, ch), i32),
            pltpu.VMEM((4, ch), i32),
            pltpu.VMEM((4, ch), f32),
            pltpu.VMEM((ch,), i32),
            pltpu.VMEM((ch, D), f32),
            pltpu.VMEM((ch, D), f32),
            pltpu.VMEM_SHARED((NP, D), f32),
            pltpu.SemaphoreType.DMA,
            pltpu.SemaphoreType.DMA,
            pltpu.SemaphoreType.DMA,
            pltpu.SemaphoreType.DMA,
            pltpu.SemaphoreType.DMA,
            pltpu.SemaphoreType.DMA,
            pltpu.SemaphoreType.DMA,
            pltpu.SemaphoreType.DMA,
        ],
    )
    def agg(t0, t1, src_hbm, dst_hbm, ew_hbm, out0, out1,
            srcb, dstb, ewb, dumb, buf0, buf1, acc,
            i0, i1, i2, i3, g0, g1, s0, s1):
        c = lax.axis_index("c")
        s = lax.axis_index("s")
        zero16 = jnp.zeros((16,), f32)
        izero16 = jnp.zeros((16,), i32)

        def zb(i, carry):
            for k in range(D // 16):
                buf0[i, pl.ds(k * 16, 16)] = zero16
                buf1[i, pl.ds(k * 16, 16)] = zero16
            return carry

        lax.fori_loop(0, ch, zb, None)
        for k in range(ch // 16):
            dumb[pl.ds(k * 16, 16)] = izero16
        for k in range(RPT // ch):
            pltpu.sync_copy(buf0, acc.at[pl.ds(s * RPT + k * ch, ch)])
        plsc.subcore_barrier()

        g = c * NS + s if edge_split else s
        bufs = (buf0, buf1)
        isem = (i0, i1, i2, i3)
        gsem = (g0, g1)
        ssem = (s0, s1)

        def run(table, out):
            ebase = g * ept

            def idx_refs(ci, q):
                off = ebase + ci * ch
                return (
                    (src_hbm.at[pl.ds(off, ch)], srcb.at[q]),
                    (dst_hbm.at[pl.ds(off, ch)], dstb.at[q]),
                    (ew_hbm.at[pl.ds(off, ch)], ewb.at[q]),
                )

            def idx_load(ci, q):
                for a, b_ in idx_refs(ci, q):
                    pltpu.async_copy(a, b_, isem[q])

            def idx_wait(ci, q):
                for a, b_ in idx_refs(ci, q):
                    pltpu.make_async_copy(a, b_, isem[q]).wait()

            def gather(ci, b, q):
                pltpu.async_copy(table.at[srcb.at[q]], bufs[b], gsem[b])

            def gather_wait(ci, b, q):
                pltpu.make_async_copy(table.at[srcb.at[q]], bufs[b],
                                      gsem[b]).wait()

            def scatter(ci, b, q):
                pltpu.async_copy(bufs[b], acc.at[dstb.at[q]], ssem[b],
                                 add=True)

            def scatter_wait(ci, b, q):
                pltpu.make_async_copy(bufs[b], acc.at[dstb.at[q]],
                                      ssem[b]).wait()

            def scale(ci, b, q):
                buf = bufs[b]
                qv = jnp.zeros((16,), i32) + q

                @plsc.parallel_loop(0, ch, unroll=2)
                def ebody(j):
                    w = plsc.load_gather(ewb, [qv, jnp.zeros((16,), i32) + j])
                    for k in range(D // 16):
                        sl = pl.ds(k * 16, 16)
                        buf[j, sl] = buf[j, sl] * w

            for q in range(3):
                idx_load(q, q)
            pltpu.async_copy(buf1, acc.at[dumb], ssem[1], add=True)
            idx_wait(0, 0)
            gather(0, 0, 0)

            def chunk_step(ci, k):
                b = k % 2
                q = k % 4
                gather_wait(ci, b, q)
                scale(ci, b, q)
                scatter(ci, b, q)
                scatter_wait(ci - 1, 1 - b, (k + 3) % 4)
                idx_wait(ci + 1, (k + 1) % 4)
                gather(ci + 1, 1 - b, (k + 1) % 4)
                idx_load(ci + 3, (k + 3) % 4)

            def quad(p, carry):
                for k in range(4):
                    chunk_step(4 * p + k, k)
                return carry

            lax.fori_loop(0, nquads, quad, None)
            for k in range(ntail):
                chunk_step(nquads * 4 + k, k)

            last = nch - 1
            idx_wait(nch + 1, (last + 2) % 4)
            idx_wait(nch + 2, (last + 3) % 4)
            gather_wait(nch, (nch) % 2, nch % 4)
            scatter_wait(last, last % 2, last % 4)

            plsc.subcore_barrier()
            for k in range(RPT // ch):
                r0 = s * RPT + k * ch
                pltpu.sync_copy(acc.at[pl.ds(r0, ch)], buf0)
                pltpu.sync_copy(buf0, out.at[pl.ds(r0, ch)])

        pl.when(c == 0)(lambda: run(t0, out0))
        pl.when(c == 1)(lambda: run(t1, out1))

    return agg


_agg_cols = _make_agg(80, False)
_agg_edges = _make_agg(40, True)


def _tc1(degp, x, h):
    def body(degp_ref, x_ref, h_ref, dinv_ref, yx_ref, yh_ref):
        deg = jnp.sum(degp_ref[...], axis=0)[:N] + 1.0
        dinv = lax.rsqrt(deg)[:, None]
        dinv_ref[...] = dinv
        yx_ref[...] = dinv * x_ref[...]
        yh_ref[...] = dinv * h_ref[...]

    return pl.pallas_call(
        body,
        out_shape=(
            jax.ShapeDtypeStruct((N, 1), f32),
            jax.ShapeDtypeStruct((N, D), f32),
            jax.ShapeDtypeStruct((N, D), f32),
        ),
    )(degp, x, h)


_BLK = 1000


def _tc2(aggx, aggh, yx, yh, dinv, h, wz, wr, wh0, bz, br):
    def body(aggx_ref, aggh_ref, yx_ref, yh_ref, dinv_ref, h_ref,
             wz_ref, wr_ref, wh0_ref, bz_ref, br_ref,
             z_ref, yrh_ref, hp_ref):
        dv = dinv_ref[...]
        a_x = dv * (aggx_ref[...] + yx_ref[...])
        a_h = dv * (aggh_ref[...] + yh_ref[...])
        wz_ = wz_ref[...]
        wr_ = wr_ref[...]
        z = jax.nn.sigmoid(
            jnp.dot(a_x, wz_[:D], preferred_element_type=f32)
            + jnp.dot(a_h, wz_[D:], preferred_element_type=f32)
            + bz_ref[...])
        r = jax.nn.sigmoid(
            jnp.dot(a_x, wr_[:D], preferred_element_type=f32)
            + jnp.dot(a_h, wr_[D:], preferred_element_type=f32)
            + br_ref[...])
        z_ref[...] = z
        yrh_ref[...] = dv * (r * h_ref[...])
        hp_ref[...] = jnp.dot(a_x, wh0_ref[...], preferred_element_type=f32)

    grid = (N // _BLK,)
    row_spec = pl.BlockSpec((_BLK, D), lambda i: (i, 0))
    return pl.pallas_call(
        body,
        grid=grid,
        in_specs=[
            row_spec, row_spec, row_spec, row_spec,
            pl.BlockSpec((_BLK, 1), lambda i: (i, 0)),
            row_spec,
            pl.BlockSpec((2 * D, D), lambda i: (0, 0)),
            pl.BlockSpec((2 * D, D), lambda i: (0, 0)),
            pl.BlockSpec((D, D), lambda i: (0, 0)),
            pl.BlockSpec((1, D), lambda i: (0, 0)),
            pl.BlockSpec((1, D), lambda i: (0, 0)),
        ],
        out_specs=(row_spec, row_spec, row_spec),
        out_shape=(
            jax.ShapeDtypeStruct((N, D), f32),
            jax.ShapeDtypeStruct((N, D), f32),
            jax.ShapeDtypeStruct((N, D), f32),
        ),
    )(aggx, aggh, yx, yh, dinv, h, wz, wr, wh0, bz, br)


def _tc3(p0, p1, yrh, dinv, hp, z, h, wh1, bh):
    def body(p0_ref, p1_ref, yrh_ref, dinv_ref, hp_ref, z_ref, h_ref,
             wh1_ref, bh_ref, out_ref):
        dv = dinv_ref[...]
        a_rh = dv * (p0_ref[...] + p1_ref[...] + yrh_ref[...])
        hc = jnp.tanh(
            hp_ref[...]
            + jnp.dot(a_rh, wh1_ref[...], preferred_element_type=f32)
            + bh_ref[...])
        zz = z_ref[...]
        out_ref[...] = zz * h_ref[...] + (1.0 - zz) * hc

    grid = (N // _BLK,)
    row_spec = pl.BlockSpec((_BLK, D), lambda i: (i, 0))
    return pl.pallas_call(
        body,
        grid=grid,
        in_specs=[
            row_spec, row_spec, row_spec,
            pl.BlockSpec((_BLK, 1), lambda i: (i, 0)),
            row_spec, row_spec, row_spec,
            pl.BlockSpec((D, D), lambda i: (0, 0)),
            pl.BlockSpec((1, D), lambda i: (0, 0)),
        ],
        out_specs=row_spec,
        out_shape=jax.ShapeDtypeStruct((N, D), f32),
    )(p0, p1, yrh, dinv, hp, z, h, wh1, bh)


def kernel(x, edge_index, edge_weight, hidden_state, W_z, b_z, W_r, b_r,
           W_h, b_h):
    src = edge_index[0]
    dst = edge_index[1]
    ew = edge_weight

    pad_i = jnp.zeros((240,), dtype=src.dtype)
    pad_f = jnp.zeros((240,), dtype=ew.dtype)
    srcp = jnp.concatenate([src, pad_i])
    dstp = jnp.concatenate([dst, pad_i])
    ewp = jnp.concatenate([ew, pad_f])

    degp = _deg_kernel(dst, ew).reshape(NC * NS, NP)
    dinv, y_x, y_h = _tc1(degp, x, hidden_state)
    aggx, aggh = _agg_cols(y_x, y_h, srcp, dstp, ewp)
    z, y_rh, hpart = _tc2(
        aggx[:N], aggh[:N], y_x, y_h, dinv, hidden_state,
        W_z, W_r, W_h[:D], b_z.reshape(1, D), b_r.reshape(1, D))
    p0, p1 = _agg_edges(y_rh, y_rh, srcp, dstp, ewp)
    h_new = _tc3(p0[:N], p1[:N], y_rh, dinv, hpart, z, hidden_state,
                 W_h[D:], b_h.reshape(1, D))
    return h_new


# speedup vs baseline: 15.1037x; 1.1114x over previous
import functools

import jax
import jax.numpy as jnp
from jax import lax
from jax.experimental import pallas as pl
from jax.experimental.pallas import tpu as pltpu
from jax.experimental.pallas import tpu_sc as plsc

N = 10000
NP = 10240
E = 320000
D = 128
NC = 2
NS = 16
CH = 80
RPT = NP // NS

f32 = jnp.float32
i32 = jnp.int32


def _sc_mesh():
    return plsc.VectorSubcoreMesh(core_axis_name="c", subcore_axis_name="s")


_EPT_DEG = E // (NC * NS)


@functools.partial(
    pl.kernel,
    out_type=jax.ShapeDtypeStruct((NC * NS * NP,), f32),
    mesh=_sc_mesh(),
    compiler_params=pltpu.CompilerParams(needs_layout_passes=False),
    scratch_types=[
        pltpu.VMEM((NP,), f32),
        pltpu.VMEM((_EPT_DEG,), i32),
        pltpu.VMEM((_EPT_DEG,), f32),
    ],
)
def _deg_kernel(dst_hbm, ew_hbm, out_hbm, deg_v, dst_v, ew_v):
    c = lax.axis_index("c")
    s = lax.axis_index("s")
    wid = c * NS + s
    zero16 = jnp.zeros((16,), f32)

    def zbody(i, carry):
        deg_v[pl.ds(i * 16, 16)] = zero16
        return carry

    lax.fori_loop(0, NP // 16, zbody, None)

    base_e = wid * _EPT_DEG
    pltpu.sync_copy(dst_hbm.at[pl.ds(base_e, _EPT_DEG)], dst_v)
    pltpu.sync_copy(ew_hbm.at[pl.ds(base_e, _EPT_DEG)], ew_v)

    @plsc.parallel_loop(0, _EPT_DEG // 16, unroll=4)
    def ebody(j):
        idx = dst_v[pl.ds(j * 16, 16)]
        val = ew_v[pl.ds(j * 16, 16)]
        plsc.addupdate_scatter(deg_v, [idx], val)

    pltpu.sync_copy(deg_v, out_hbm.at[pl.ds(wid * NP, NP)])


def _make_agg(ch, edge_split):
    ngroups = NC * NS if edge_split else NS
    ept = E // ngroups
    nch = ept // ch
    nquads = nch // 4
    ntail = nch % 4

    @functools.partial(
        pl.kernel,
        out_type=(
            jax.ShapeDtypeStruct((NP, D), f32),
            jax.ShapeDtypeStruct((NP, D), f32),
        ),
        mesh=_sc_mesh(),
        compiler_params=pltpu.CompilerParams(needs_layout_passes=False),
        scratch_types=[
            pltpu.VMEM((4, ch), i32),
            pltpu.VMEM((4, ch), i32),
            pltpu.VMEM((4, ch), f32),
            pltpu.VMEM((ch,), i32),
            pltpu.VMEM((ch, D), f32),
            pltpu.VMEM((ch, D), f32),
            pltpu.VMEM_SHARED((NP, D), f32),
            pltpu.SemaphoreType.DMA,
            pltpu.SemaphoreType.DMA,
            pltpu.SemaphoreType.DMA,
            pltpu.SemaphoreType.DMA,
            pltpu.SemaphoreType.DMA,
            pltpu.SemaphoreType.DMA,
            pltpu.SemaphoreType.DMA,
            pltpu.SemaphoreType.DMA,
        ],
    )
    def agg(t0, t1, src_hbm, dst_hbm, ew_hbm, out0, out1,
            srcb, dstb, ewb, dumb, buf0, buf1, acc,
            i0, i1, i2, i3, g0, g1, s0, s1):
        c = lax.axis_index("c")
        s = lax.axis_index("s")
        zero16 = jnp.zeros((16,), f32)
        izero16 = jnp.zeros((16,), i32)

        def zb(i, carry):
            for k in range(D // 16):
                buf0[i, pl.ds(k * 16, 16)] = zero16
                buf1[i, pl.ds(k * 16, 16)] = zero16
            return carry

        lax.fori_loop(0, ch, zb, None)
        for k in range(ch // 16):
            dumb[pl.ds(k * 16, 16)] = izero16
        for k in range(RPT // ch):
            pltpu.sync_copy(buf0, acc.at[pl.ds(s * RPT + k * ch, ch)])
        plsc.subcore_barrier()

        g = c * NS + s if edge_split else s
        bufs = (buf0, buf1)
        isem = (i0, i1, i2, i3)
        gsem = (g0, g1)
        ssem = (s0, s1)

        def run(table, out):
            ebase = g * ept

            def idx_refs(ci, q):
                off = ebase + ci * ch
                return (
                    (src_hbm.at[pl.ds(off, ch)], srcb.at[q]),
                    (dst_hbm.at[pl.ds(off, ch)], dstb.at[q]),
                    (ew_hbm.at[pl.ds(off, ch)], ewb.at[q]),
                )

            def idx_load(ci, q):
                for a, b_ in idx_refs(ci, q):
                    pltpu.async_copy(a, b_, isem[q])

            def idx_wait(ci, q):
                for a, b_ in idx_refs(ci, q):
                    pltpu.make_async_copy(a, b_, isem[q]).wait()

            def gather(ci, b, q):
                pltpu.async_copy(table.at[srcb.at[q]], bufs[b], gsem[b])

            def gather_wait(ci, b, q):
                pltpu.make_async_copy(table.at[srcb.at[q]], bufs[b],
                                      gsem[b]).wait()

            def scatter(ci, b, q):
                pltpu.async_copy(bufs[b], acc.at[dstb.at[q]], ssem[b],
                                 add=True)

            def scatter_wait(ci, b, q):
                pltpu.make_async_copy(bufs[b], acc.at[dstb.at[q]],
                                      ssem[b]).wait()

            def scale(ci, b, q):
                buf = bufs[b]
                qv = jnp.zeros((16,), i32) + q

                @plsc.parallel_loop(0, ch, unroll=2)
                def ebody(j):
                    w = plsc.load_gather(ewb, [qv, jnp.zeros((16,), i32) + j])
                    for k in range(D // 16):
                        sl = pl.ds(k * 16, 16)
                        buf[j, sl] = buf[j, sl] * w

            for q in range(3):
                idx_load(q, q)
            pltpu.async_copy(buf1, acc.at[dumb], ssem[1], add=True)
            idx_wait(0, 0)
            gather(0, 0, 0)

            def chunk_step(ci, k):
                b = k % 2
                q = k % 4
                gather_wait(ci, b, q)
                scale(ci, b, q)
                scatter(ci, b, q)
                scatter_wait(ci - 1, 1 - b, (k + 3) % 4)
                idx_wait(ci + 1, (k + 1) % 4)
                gather(ci + 1, 1 - b, (k + 1) % 4)
                idx_load(ci + 3, (k + 3) % 4)

            def quad(p, carry):
                for k in range(4):
                    chunk_step(4 * p + k, k)
                return carry

            lax.fori_loop(0, nquads, quad, None)
            for k in range(ntail):
                chunk_step(nquads * 4 + k, k)

            last = nch - 1
            idx_wait(nch + 1, (last + 2) % 4)
            idx_wait(nch + 2, (last + 3) % 4)
            gather_wait(nch, (nch) % 2, nch % 4)
            scatter_wait(last, last % 2, last % 4)

            plsc.subcore_barrier()
            pltpu.sync_copy(acc.at[pl.ds(s * RPT, RPT)],
                            out.at[pl.ds(s * RPT, RPT)])

        pl.when(c == 0)(lambda: run(t0, out0))
        pl.when(c == 1)(lambda: run(t1, out1))

    return agg


_agg_cols = _make_agg(80, False)
_agg_edges = _make_agg(80, True)


def _tc1(degp, x, h):
    def body(degp_ref, x_ref, h_ref, dinv_ref, yx_ref, yh_ref):
        deg = jnp.sum(degp_ref[...], axis=0)[:N] + 1.0
        dinv = lax.rsqrt(deg)[:, None]
        dinv_ref[...] = dinv
        yx_ref[...] = dinv * x_ref[...]
        yh_ref[...] = dinv * h_ref[...]

    return pl.pallas_call(
        body,
        out_shape=(
            jax.ShapeDtypeStruct((N, 1), f32),
            jax.ShapeDtypeStruct((N, D), f32),
            jax.ShapeDtypeStruct((N, D), f32),
        ),
    )(degp, x, h)


_BLK = 1000


def _tc2(aggx, aggh, yx, yh, dinv, h, wz, wr, wh0, bz, br):
    def body(aggx_ref, aggh_ref, yx_ref, yh_ref, dinv_ref, h_ref,
             wz_ref, wr_ref, wh0_ref, bz_ref, br_ref,
             z_ref, yrh_ref, hp_ref):
        dv = dinv_ref[...]
        a_x = dv * (aggx_ref[...] + yx_ref[...])
        a_h = dv * (aggh_ref[...] + yh_ref[...])
        wz_ = wz_ref[...]
        wr_ = wr_ref[...]
        z = jax.nn.sigmoid(
            jnp.dot(a_x, wz_[:D], preferred_element_type=f32)
            + jnp.dot(a_h, wz_[D:], preferred_element_type=f32)
            + bz_ref[...])
        r = jax.nn.sigmoid(
            jnp.dot(a_x, wr_[:D], preferred_element_type=f32)
            + jnp.dot(a_h, wr_[D:], preferred_element_type=f32)
            + br_ref[...])
        z_ref[...] = z
        yrh_ref[...] = dv * (r * h_ref[...])
        hp_ref[...] = jnp.dot(a_x, wh0_ref[...], preferred_element_type=f32)

    grid = (N // _BLK,)
    row_spec = pl.BlockSpec((_BLK, D), lambda i: (i, 0))
    return pl.pallas_call(
        body,
        grid=grid,
        in_specs=[
            row_spec, row_spec, row_spec, row_spec,
            pl.BlockSpec((_BLK, 1), lambda i: (i, 0)),
            row_spec,
            pl.BlockSpec((2 * D, D), lambda i: (0, 0)),
            pl.BlockSpec((2 * D, D), lambda i: (0, 0)),
            pl.BlockSpec((D, D), lambda i: (0, 0)),
            pl.BlockSpec((1, D), lambda i: (0, 0)),
            pl.BlockSpec((1, D), lambda i: (0, 0)),
        ],
        out_specs=(row_spec, row_spec, row_spec),
        out_shape=(
            jax.ShapeDtypeStruct((N, D), f32),
            jax.ShapeDtypeStruct((N, D), f32),
            jax.ShapeDtypeStruct((N, D), f32),
        ),
    )(aggx, aggh, yx, yh, dinv, h, wz, wr, wh0, bz, br)


def _tc3(p0, p1, yrh, dinv, hp, z, h, wh1, bh):
    def body(p0_ref, p1_ref, yrh_ref, dinv_ref, hp_ref, z_ref, h_ref,
             wh1_ref, bh_ref, out_ref):
        dv = dinv_ref[...]
        a_rh = dv * (p0_ref[...] + p1_ref[...] + yrh_ref[...])
        hc = jnp.tanh(
            hp_ref[...]
            + jnp.dot(a_rh, wh1_ref[...], preferred_element_type=f32)
            + bh_ref[...])
        zz = z_ref[...]
        out_ref[...] = zz * h_ref[...] + (1.0 - zz) * hc

    grid = (N // _BLK,)
    row_spec = pl.BlockSpec((_BLK, D), lambda i: (i, 0))
    return pl.pallas_call(
        body,
        grid=grid,
        in_specs=[
            row_spec, row_spec, row_spec,
            pl.BlockSpec((_BLK, 1), lambda i: (i, 0)),
            row_spec, row_spec, row_spec,
            pl.BlockSpec((D, D), lambda i: (0, 0)),
            pl.BlockSpec((1, D), lambda i: (0, 0)),
        ],
        out_specs=row_spec,
        out_shape=jax.ShapeDtypeStruct((N, D), f32),
    )(p0, p1, yrh, dinv, hp, z, h, wh1, bh)


def kernel(x, edge_index, edge_weight, hidden_state, W_z, b_z, W_r, b_r,
           W_h, b_h):
    src = edge_index[0]
    dst = edge_index[1]
    ew = edge_weight

    pad_i = jnp.zeros((240,), dtype=src.dtype)
    pad_f = jnp.zeros((240,), dtype=ew.dtype)
    srcp = jnp.concatenate([src, pad_i])
    dstp = jnp.concatenate([dst, pad_i])
    ewp = jnp.concatenate([ew, pad_f])

    degp = _deg_kernel(dst, ew).reshape(NC * NS, NP)
    dinv, y_x, y_h = _tc1(degp, x, hidden_state)
    aggx, aggh = _agg_cols(y_x, y_h, srcp, dstp, ewp)
    z, y_rh, hpart = _tc2(
        aggx[:N], aggh[:N], y_x, y_h, dinv, hidden_state,
        W_z, W_r, W_h[:D], b_z.reshape(1, D), b_r.reshape(1, D))
    p0, p1 = _agg_edges(y_rh, y_rh, srcp, dstp, ewp)
    h_new = _tc3(p0[:N], p1[:N], y_rh, dinv, hpart, z, hidden_state,
                 W_h[D:], b_h.reshape(1, D))
    return h_new
